# baseline (speedup 1.0000x reference)
import functools

import jax
import jax.numpy as jnp
from jax import lax
from jax.experimental import pallas as pl
from jax.experimental.pallas import tpu as pltpu
from jax.experimental.pallas import tpu_sc as plsc

_K = 8
_DD = 16
_NC, _NS = 2, 16
_NW = _NC * _NS
_ITERS = 6
_LOG2E = 1.4426950408889634


def _seg_matrix(d, dtype):
    lane = lax.broadcasted_iota(jnp.int32, (d, _K), 0)
    cap = lax.broadcasted_iota(jnp.int32, (d, _K), 1)
    return (lane // _DD == cap).astype(dtype)


def _seg2(d, lo, dtype):
    lane = lax.broadcasted_iota(jnp.int32, (d, 2 * _K), 0)
    col = lax.broadcasted_iota(jnp.int32, (d, 2 * _K), 1)
    cap = (lane % 64) // _DD + (0 if lo else _K // 2) + _K * (lane // 64)
    return (cap == col).astype(dtype)


def _unpack_lo(v):
    return lax.bitcast_convert_type(v << 16, jnp.float32)


def _unpack_hi(v):
    return lax.bitcast_convert_type(v & jnp.int32(-65536), jnp.float32)


def _rne_bf16_bits(f):
    b = lax.bitcast_convert_type(f, jnp.int32)
    return (b + 0x7FFF + ((b >> 16) & 1)) >> 16


def _prep_body(x_ref, nbr_ref, t_ref, nt_ref):
    x = x_ref[...]
    d = x.shape[-1]
    s_mat = _seg_matrix(d, x.dtype)
    sq = jnp.dot(x * x, s_mat, preferred_element_type=jnp.float32)
    scale = lax.rsqrt(jnp.maximum(sq, 1e-24))
    xn = x * jnp.dot(scale, s_mat.T, preferred_element_type=jnp.float32)
    lo = _rne_bf16_bits(xn[:, : d // 2])
    hi = _rne_bf16_bits(xn[:, d // 2:])
    t_ref[...] = (hi << 16) | (lo & 0xFFFF)

    @pl.when(pl.program_id(0) == 0)
    def _():
        nt_ref[...] = nbr_ref[...].T


def _routing_body(z_ref, x_ref, o_ref):
    z32 = z_ref[...]
    x32 = x_ref[...]
    m, bh, d = z32.shape
    z_lo, z_hi = _unpack_lo(z32), _unpack_hi(z32)
    x_lo, x_hi = _unpack_lo(x32), _unpack_hi(x32)
    s_lo = _seg2(d, True, jnp.float32)
    s_hi = _seg2(d, False, jnp.float32)
    s16_lo = s_lo.astype(jnp.bfloat16)
    s16_hi = s_hi.astype(jnp.bfloat16)
    sl16_lo = (s_lo * _LOG2E).astype(jnp.bfloat16)
    sl16_hi = (s_hi * _LOG2E).astype(jnp.bfloat16)
    r16 = lax.broadcasted_iota(jnp.int32, (2 * _K, 2 * _K), 0)
    c16 = lax.broadcasted_iota(jnp.int32, (2 * _K, 2 * _K), 1)
    b16 = (r16 // _K == c16 // _K).astype(jnp.float32)
    zf_lo = z_lo.reshape(m * bh, d)
    zf_hi = z_hi.reshape(m * bh, d)

    def cap_norm(u_lo, u_hi):
        sq = (jnp.dot(u_lo * u_lo, s_lo, preferred_element_type=jnp.float32)
              + jnp.dot(u_hi * u_hi, s_hi, preferred_element_type=jnp.float32))
        scale = lax.rsqrt(jnp.maximum(sq, 1e-24))
        return (u_lo * jnp.dot(scale, s_lo.T,
                               preferred_element_type=jnp.float32),
                u_hi * jnp.dot(scale, s_hi.T,
                               preferred_element_type=jnp.float32))

    u_lo = jnp.sum(z_lo, axis=0) * (1.0 / _K) + x_lo
    u_hi = jnp.sum(z_hi, axis=0) * (1.0 / _K) + x_hi
    u_lo, u_hi = cap_norm(u_lo, u_hi)
    for it in range(1, _ITERS):
        t_lo = (z_lo * u_lo[None, :, :]).reshape(m * bh, d).astype(jnp.bfloat16)
        t_hi = (z_hi * u_hi[None, :, :]).reshape(m * bh, d).astype(jnp.bfloat16)
        p = (jnp.dot(t_lo, sl16_lo, preferred_element_type=jnp.float32)
             + jnp.dot(t_hi, sl16_hi, preferred_element_type=jnp.float32))
        e = jnp.exp2(p)
        den = jnp.dot(e, b16, preferred_element_type=jnp.float32)
        pn = (e / den).astype(jnp.bfloat16)
        pe_lo = jnp.dot(pn, s16_lo.T, preferred_element_type=jnp.float32)
        pe_hi = jnp.dot(pn, s16_hi.T, preferred_element_type=jnp.float32)
        u_lo = jnp.sum((zf_lo * pe_lo).reshape(m, bh, d), axis=0) + x_lo
        u_hi = jnp.sum((zf_hi * pe_hi).reshape(m, bh, d), axis=0) + x_hi
        if it < _ITERS - 1:
            u_lo, u_hi = cap_norm(u_lo, u_hi)
    h = d // 2
    o_ref[...] = jnp.concatenate(
        [u_lo[:, :h], u_hi[:, :h], u_lo[:, h:], u_hi[:, h:]], axis=1)


def _prep_call(x, nbr2):
    n, d = x.shape
    m = nbr2.shape[1]
    bn = 2000
    return pl.pallas_call(
        _prep_body,
        grid=(n // bn,),
        in_specs=[pl.BlockSpec((bn, d), lambda i: (i, 0)),
                  pl.BlockSpec((n, m), lambda i: (0, 0))],
        out_specs=[pl.BlockSpec((bn, d // 2), lambda i: (i, 0)),
                   pl.BlockSpec((m, n), lambda i: (0, 0))],
        out_shape=[jax.ShapeDtypeStruct((n, d // 2), jnp.int32),
                   jax.ShapeDtypeStruct((m, n), nbr2.dtype)],
    )(x, nbr2)


def _routing_call(z32v, x32v):
    m, n2, d = z32v.shape
    bh = 200
    return pl.pallas_call(
        _routing_body,
        grid=(n2 // bh,),
        in_specs=[
            pl.BlockSpec((m, bh, d), lambda i: (0, i, 0)),
            pl.BlockSpec((bh, d), lambda i: (i, 0)),
        ],
        out_specs=pl.BlockSpec((bh, 2 * d), lambda i: (i, 0)),
        out_shape=jax.ShapeDtypeStruct((n2, 2 * d), jnp.float32),
        compiler_params=pltpu.CompilerParams(
            vmem_limit_bytes=120 * 1024 * 1024),
    )(z32v, x32v)


def _gather_call(table, idx, n, lo, hi, chunk):
    per_w = hi - lo
    e_total = per_w * _NW
    steps = per_w // chunk
    d = table.shape[1]
    mesh = plsc.VectorSubcoreMesh(core_axis_name="c", subcore_axis_name="s")

    @functools.partial(
        pl.kernel,
        out_type=jax.ShapeDtypeStruct((e_total, d), table.dtype),
        mesh=mesh,
        scratch_types=[
            pltpu.VMEM((chunk,), jnp.int32),
            pltpu.VMEM((chunk,), jnp.int32),
            pltpu.VMEM((chunk, d), table.dtype),
            pltpu.VMEM((chunk, d), table.dtype),
            pltpu.SemaphoreType.DMA,
            pltpu.SemaphoreType.DMA,
        ],
        compiler_params=pltpu.CompilerParams(use_tc_tiling_on_sc=False),
    )
    def gather_kernel(table_hbm, idx_hbm, out_hbm,
                      idx_v0, idx_v1, rows_v0, rows_v1, sem0, sem1):
        wid = lax.axis_index("s") * _NC + lax.axis_index("c")
        src_base = wid * n + lo
        dst_base = wid * per_w
        idx_b, rows_b, sem_b = (idx_v0, idx_v1), (rows_v0, rows_v1), (sem0, sem1)

        def start(i):
            b = i % 2
            pltpu.sync_copy(idx_hbm.at[pl.ds(src_base + i * chunk, chunk)],
                            idx_b[b])
            return pltpu.async_copy(table_hbm.at[idx_b[b]], rows_b[b], sem_b[b])

        pending = start(0)
        for i in range(steps):
            nxt = start(i + 1) if i + 1 < steps else None
            pending.wait()
            pltpu.sync_copy(rows_b[i % 2],
                            out_hbm.at[pl.ds(dst_base + i * chunk, chunk)])
            pending = nxt

    return gather_kernel(table, idx)


def kernel(x, neighbors, max_iter, last_layer):
    del max_iter, last_layer
    n, d = x.shape
    m = neighbors.shape[0] // n
    table_body, nbr_t = _prep_call(x, neighbors.reshape(n, m))
    table = jnp.concatenate(
        [table_body, jnp.zeros((8, d // 2), dtype=jnp.int32)], axis=0)
    na = 2400
    x32 = table_body.reshape(n // 2, d)
    us = []
    nbr_flat = nbr_t.reshape(m * n)
    for lo, hi, chunk in ((0, na, 800), (na, n, 760)):
        z32 = _gather_call(table, nbr_flat, n, lo, hi, chunk)
        z32v = z32.reshape(m, (hi - lo) // 2, d)
        us.append(_routing_call(z32v, x32[lo // 2: hi // 2]))
    return jnp.concatenate(us, axis=0).reshape(n, d)


# speedup vs baseline: 1.0090x; 1.0090x over previous
import functools

import jax
import jax.numpy as jnp
from jax import lax
from jax.experimental import pallas as pl
from jax.experimental.pallas import tpu as pltpu
from jax.experimental.pallas import tpu_sc as plsc

_K = 8
_DD = 16
_NC, _NS = 2, 16
_NW = _NC * _NS
_ITERS = 6
_LOG2E = 1.4426950408889634


def _seg_matrix(d, dtype):
    lane = lax.broadcasted_iota(jnp.int32, (d, _K), 0)
    cap = lax.broadcasted_iota(jnp.int32, (d, _K), 1)
    return (lane // _DD == cap).astype(dtype)


def _seg2(d, lo, dtype):
    lane = lax.broadcasted_iota(jnp.int32, (d, 2 * _K), 0)
    col = lax.broadcasted_iota(jnp.int32, (d, 2 * _K), 1)
    cap = (lane % 64) // _DD + (0 if lo else _K // 2) + _K * (lane // 64)
    return (cap == col).astype(dtype)


def _unpack_lo(v):
    return lax.bitcast_convert_type(v << 16, jnp.float32)


def _unpack_hi(v):
    return lax.bitcast_convert_type(v & jnp.int32(-65536), jnp.float32)


def _rne_bf16_bits(f):
    b = lax.bitcast_convert_type(f, jnp.int32)
    return (b + 0x7FFF + ((b >> 16) & 1)) >> 16


def _prep_body(x_ref, nbr_ref, t_ref, nt_ref):
    x = x_ref[...]
    d = x.shape[-1]
    s_mat = _seg_matrix(d, x.dtype)
    sq = jnp.dot(x * x, s_mat, preferred_element_type=jnp.float32)
    scale = lax.rsqrt(jnp.maximum(sq, 1e-24))
    xn = x * jnp.dot(scale, s_mat.T, preferred_element_type=jnp.float32)
    lo = _rne_bf16_bits(xn[:, : d // 2])
    hi = _rne_bf16_bits(xn[:, d // 2:])
    t_ref[...] = (hi << 16) | (lo & 0xFFFF)

    @pl.when(pl.program_id(0) == 0)
    def _():
        nt_ref[...] = nbr_ref[...].T


def _routing_body(z_ref, x_ref, o_ref):
    z32 = z_ref[...]
    x32 = x_ref[...]
    m, bh, d = z32.shape
    z_lo, z_hi = _unpack_lo(z32), _unpack_hi(z32)
    x_lo, x_hi = _unpack_lo(x32), _unpack_hi(x32)
    s_lo = _seg2(d, True, jnp.float32)
    s_hi = _seg2(d, False, jnp.float32)
    s16_lo = s_lo.astype(jnp.bfloat16)
    s16_hi = s_hi.astype(jnp.bfloat16)
    sl16_lo = (s_lo * _LOG2E).astype(jnp.bfloat16)
    sl16_hi = (s_hi * _LOG2E).astype(jnp.bfloat16)
    r16 = lax.broadcasted_iota(jnp.int32, (2 * _K, 2 * _K), 0)
    c16 = lax.broadcasted_iota(jnp.int32, (2 * _K, 2 * _K), 1)
    b16 = (r16 // _K == c16 // _K).astype(jnp.float32)
    zf_lo = z_lo.reshape(m * bh, d)
    zf_hi = z_hi.reshape(m * bh, d)

    def cap_norm(u_lo, u_hi):
        sq = (jnp.dot(u_lo * u_lo, s_lo, preferred_element_type=jnp.float32)
              + jnp.dot(u_hi * u_hi, s_hi, preferred_element_type=jnp.float32))
        scale = lax.rsqrt(jnp.maximum(sq, 1e-24))
        return (u_lo * jnp.dot(scale, s_lo.T,
                               preferred_element_type=jnp.float32),
                u_hi * jnp.dot(scale, s_hi.T,
                               preferred_element_type=jnp.float32))

    u_lo = jnp.sum(z_lo, axis=0) * (1.0 / _K) + x_lo
    u_hi = jnp.sum(z_hi, axis=0) * (1.0 / _K) + x_hi
    u_lo, u_hi = cap_norm(u_lo, u_hi)
    for it in range(1, _ITERS):
        t_lo = (z_lo * u_lo[None, :, :]).reshape(m * bh, d).astype(jnp.bfloat16)
        t_hi = (z_hi * u_hi[None, :, :]).reshape(m * bh, d).astype(jnp.bfloat16)
        p = (jnp.dot(t_lo, sl16_lo, preferred_element_type=jnp.float32)
             + jnp.dot(t_hi, sl16_hi, preferred_element_type=jnp.float32))
        e = jnp.exp2(p)
        den = jnp.dot(e, b16, preferred_element_type=jnp.float32)
        pn = (e / den).astype(jnp.bfloat16)
        pe_lo = jnp.dot(pn, s16_lo.T, preferred_element_type=jnp.float32)
        pe_hi = jnp.dot(pn, s16_hi.T, preferred_element_type=jnp.float32)
        u_lo = jnp.sum((zf_lo * pe_lo).reshape(m, bh, d), axis=0) + x_lo
        u_hi = jnp.sum((zf_hi * pe_hi).reshape(m, bh, d), axis=0) + x_hi
        if it < _ITERS - 1:
            u_lo, u_hi = cap_norm(u_lo, u_hi)
    h = d // 2
    o_ref[...] = jnp.concatenate(
        [u_lo[:, :h], u_hi[:, :h], u_lo[:, h:], u_hi[:, h:]], axis=1)


def _prep_call(x, nbr2):
    n, d = x.shape
    m = nbr2.shape[1]
    bn = 2000
    return pl.pallas_call(
        _prep_body,
        grid=(n // bn,),
        in_specs=[pl.BlockSpec((bn, d), lambda i: (i, 0)),
                  pl.BlockSpec((n, m), lambda i: (0, 0))],
        out_specs=[pl.BlockSpec((bn, d // 2), lambda i: (i, 0)),
                   pl.BlockSpec((m, n), lambda i: (0, 0))],
        out_shape=[jax.ShapeDtypeStruct((n, d // 2), jnp.int32),
                   jax.ShapeDtypeStruct((m, n), nbr2.dtype)],
    )(x, nbr2)


def _routing_call(z32v, x32v):
    m, n2, d = z32v.shape
    bh = 200
    return pl.pallas_call(
        _routing_body,
        grid=(n2 // bh,),
        in_specs=[
            pl.BlockSpec((m, bh, d), lambda i: (0, i, 0)),
            pl.BlockSpec((bh, d), lambda i: (i, 0)),
        ],
        out_specs=pl.BlockSpec((bh, 2 * d), lambda i: (i, 0)),
        out_shape=jax.ShapeDtypeStruct((n2, 2 * d), jnp.float32),
        compiler_params=pltpu.CompilerParams(
            vmem_limit_bytes=120 * 1024 * 1024),
    )(z32v, x32v)


def _gather_call(table, idx, n, lo, hi, chunk):
    per_w = hi - lo
    e_total = per_w * _NW
    steps = per_w // chunk
    d = table.shape[1]
    mesh = plsc.VectorSubcoreMesh(core_axis_name="c", subcore_axis_name="s")

    @functools.partial(
        pl.kernel,
        out_type=jax.ShapeDtypeStruct((e_total, d), table.dtype),
        mesh=mesh,
        scratch_types=[
            pltpu.VMEM((chunk,), jnp.int32),
            pltpu.VMEM((chunk,), jnp.int32),
            pltpu.VMEM((chunk, d), table.dtype),
            pltpu.VMEM((chunk, d), table.dtype),
            pltpu.SemaphoreType.DMA,
            pltpu.SemaphoreType.DMA,
        ],
        compiler_params=pltpu.CompilerParams(use_tc_tiling_on_sc=False),
    )
    def gather_kernel(table_hbm, idx_hbm, out_hbm,
                      idx_v0, idx_v1, rows_v0, rows_v1, sem0, sem1):
        wid = lax.axis_index("s") * _NC + lax.axis_index("c")
        src_base = wid * n + lo
        dst_base = wid * per_w
        idx_b, rows_b, sem_b = (idx_v0, idx_v1), (rows_v0, rows_v1), (sem0, sem1)

        def start(i):
            b = i % 2
            pltpu.sync_copy(idx_hbm.at[pl.ds(src_base + i * chunk, chunk)],
                            idx_b[b])
            return pltpu.async_copy(table_hbm.at[idx_b[b]], rows_b[b], sem_b[b])

        pending = start(0)
        for i in range(steps):
            nxt = start(i + 1) if i + 1 < steps else None
            pending.wait()
            pltpu.sync_copy(rows_b[i % 2],
                            out_hbm.at[pl.ds(dst_base + i * chunk, chunk)])
            pending = nxt

    return gather_kernel(table, idx)


def kernel(x, neighbors, max_iter, last_layer):
    del max_iter, last_layer
    n, d = x.shape
    m = neighbors.shape[0] // n
    table_body, nbr_t = _prep_call(x, neighbors.reshape(n, m))
    table = jnp.concatenate(
        [table_body, jnp.zeros((8, d // 2), dtype=jnp.int32)], axis=0)
    na = 2400
    x32 = table_body.reshape(n // 2, d)
    us = []
    for lo, hi, chunk in ((0, na, 800), (na, n, 760)):
        nbr_c = nbr_t[:, lo:hi].reshape(m * (hi - lo))
        z32 = _gather_call(table, nbr_c, hi - lo, 0, hi - lo, chunk)
        z32v = z32.reshape(m, (hi - lo) // 2, d)
        us.append(_routing_call(z32v, x32[lo // 2: hi // 2]))
    return jnp.concatenate(us, axis=0).reshape(n, d)


# speedup vs baseline: 1.0111x; 1.0020x over previous
import functools

import jax
import jax.numpy as jnp
from jax import lax
from jax.experimental import pallas as pl
from jax.experimental.pallas import tpu as pltpu
from jax.experimental.pallas import tpu_sc as plsc

_K = 8
_DD = 16
_NC, _NS = 2, 16
_NW = _NC * _NS
_ITERS = 6
_LOG2E = 1.4426950408889634


def _seg_matrix(d, dtype):
    lane = lax.broadcasted_iota(jnp.int32, (d, _K), 0)
    cap = lax.broadcasted_iota(jnp.int32, (d, _K), 1)
    return (lane // _DD == cap).astype(dtype)


def _seg2(d, lo, dtype):
    lane = lax.broadcasted_iota(jnp.int32, (d, 2 * _K), 0)
    col = lax.broadcasted_iota(jnp.int32, (d, 2 * _K), 1)
    cap = (lane % 64) // _DD + (0 if lo else _K // 2) + _K * (lane // 64)
    return (cap == col).astype(dtype)


def _unpack_lo(v):
    return lax.bitcast_convert_type(v << 16, jnp.float32)


def _unpack_hi(v):
    return lax.bitcast_convert_type(v & jnp.int32(-65536), jnp.float32)


def _rne_bf16_bits(f):
    b = lax.bitcast_convert_type(f, jnp.int32)
    return (b + 0x7FFF + ((b >> 16) & 1)) >> 16


def _prep_body(x_ref, nbr_ref, t_ref, nt_ref):
    x = x_ref[...]
    d = x.shape[-1]
    s_mat = _seg_matrix(d, x.dtype)
    sq = jnp.dot(x * x, s_mat, preferred_element_type=jnp.float32)
    scale = lax.rsqrt(jnp.maximum(sq, 1e-24))
    xn = x * jnp.dot(scale, s_mat.T, preferred_element_type=jnp.float32)
    lo = _rne_bf16_bits(xn[:, : d // 2])
    hi = _rne_bf16_bits(xn[:, d // 2:])
    t_ref[...] = (hi << 16) | (lo & 0xFFFF)

    @pl.when(pl.program_id(0) == 0)
    def _():
        nt_ref[...] = nbr_ref[...].T


def _routing_body(z_ref, x_ref, o_ref):
    z32 = z_ref[...]
    x32 = x_ref[...]
    m, bh, d = z32.shape
    z_lo, z_hi = _unpack_lo(z32), _unpack_hi(z32)
    x_lo, x_hi = _unpack_lo(x32), _unpack_hi(x32)
    s_lo = _seg2(d, True, jnp.float32)
    s_hi = _seg2(d, False, jnp.float32)
    s16_lo = s_lo.astype(jnp.bfloat16)
    s16_hi = s_hi.astype(jnp.bfloat16)
    sl16_lo = (s_lo * _LOG2E).astype(jnp.bfloat16)
    sl16_hi = (s_hi * _LOG2E).astype(jnp.bfloat16)
    r16 = lax.broadcasted_iota(jnp.int32, (2 * _K, 2 * _K), 0)
    c16 = lax.broadcasted_iota(jnp.int32, (2 * _K, 2 * _K), 1)
    b16 = (r16 // _K == c16 // _K).astype(jnp.float32)
    zf_lo = z_lo.reshape(m * bh, d)
    zf_hi = z_hi.reshape(m * bh, d)

    def cap_norm(u_lo, u_hi):
        sq = (jnp.dot(u_lo * u_lo, s_lo, preferred_element_type=jnp.float32)
              + jnp.dot(u_hi * u_hi, s_hi, preferred_element_type=jnp.float32))
        scale = lax.rsqrt(jnp.maximum(sq, 1e-24))
        return (u_lo * jnp.dot(scale, s_lo.T,
                               preferred_element_type=jnp.float32),
                u_hi * jnp.dot(scale, s_hi.T,
                               preferred_element_type=jnp.float32))

    u_lo = jnp.sum(z_lo, axis=0) * (1.0 / _K) + x_lo
    u_hi = jnp.sum(z_hi, axis=0) * (1.0 / _K) + x_hi
    u_lo, u_hi = cap_norm(u_lo, u_hi)
    for it in range(1, _ITERS):
        t_lo = (z_lo * u_lo[None, :, :]).reshape(m * bh, d).astype(jnp.bfloat16)
        t_hi = (z_hi * u_hi[None, :, :]).reshape(m * bh, d).astype(jnp.bfloat16)
        p = (jnp.dot(t_lo, sl16_lo, preferred_element_type=jnp.float32)
             + jnp.dot(t_hi, sl16_hi, preferred_element_type=jnp.float32))
        e = jnp.exp2(p)
        den = jnp.dot(e, b16, preferred_element_type=jnp.float32)
        pn = (e / den).astype(jnp.bfloat16)
        pe_lo = jnp.dot(pn, s16_lo.T, preferred_element_type=jnp.float32)
        pe_hi = jnp.dot(pn, s16_hi.T, preferred_element_type=jnp.float32)
        u_lo = jnp.sum((zf_lo * pe_lo).reshape(m, bh, d), axis=0) + x_lo
        u_hi = jnp.sum((zf_hi * pe_hi).reshape(m, bh, d), axis=0) + x_hi
        if it < _ITERS - 1:
            u_lo, u_hi = cap_norm(u_lo, u_hi)
    h = d // 2
    o_ref[...] = jnp.concatenate(
        [u_lo[:, :h], u_hi[:, :h], u_lo[:, h:], u_hi[:, h:]], axis=1)


def _prep_call(x, nbr2):
    n, d = x.shape
    m = nbr2.shape[1]
    bn = 2000
    return pl.pallas_call(
        _prep_body,
        grid=(n // bn,),
        in_specs=[pl.BlockSpec((bn, d), lambda i: (i, 0)),
                  pl.BlockSpec((n, m), lambda i: (0, 0))],
        out_specs=[pl.BlockSpec((bn, d // 2), lambda i: (i, 0)),
                   pl.BlockSpec((m, n), lambda i: (0, 0))],
        out_shape=[jax.ShapeDtypeStruct((n, d // 2), jnp.int32),
                   jax.ShapeDtypeStruct((m, n), nbr2.dtype)],
    )(x, nbr2)


def _routing_call(z32v, x32v):
    m, n2, d = z32v.shape
    bh = 200
    return pl.pallas_call(
        _routing_body,
        grid=(n2 // bh,),
        in_specs=[
            pl.BlockSpec((m, bh, d), lambda i: (0, i, 0)),
            pl.BlockSpec((bh, d), lambda i: (i, 0)),
        ],
        out_specs=pl.BlockSpec((bh, 2 * d), lambda i: (i, 0)),
        out_shape=jax.ShapeDtypeStruct((n2, 2 * d), jnp.float32),
        compiler_params=pltpu.CompilerParams(
            vmem_limit_bytes=120 * 1024 * 1024),
    )(z32v, x32v)


def _gather_call(table, idx, n, lo, hi, chunk):
    per_w = hi - lo
    e_total = per_w * _NW
    steps = per_w // chunk
    d = table.shape[1]
    mesh = plsc.VectorSubcoreMesh(core_axis_name="c", subcore_axis_name="s")

    @functools.partial(
        pl.kernel,
        out_type=jax.ShapeDtypeStruct((e_total, d), table.dtype),
        mesh=mesh,
        scratch_types=[
            pltpu.VMEM((chunk,), jnp.int32),
            pltpu.VMEM((chunk,), jnp.int32),
            pltpu.VMEM((chunk, d), table.dtype),
            pltpu.VMEM((chunk, d), table.dtype),
            pltpu.SemaphoreType.DMA,
            pltpu.SemaphoreType.DMA,
        ],
        compiler_params=pltpu.CompilerParams(use_tc_tiling_on_sc=False),
    )
    def gather_kernel(table_hbm, idx_hbm, out_hbm,
                      idx_v0, idx_v1, rows_v0, rows_v1, sem0, sem1):
        wid = lax.axis_index("s") * _NC + lax.axis_index("c")
        src_base = wid * n + lo
        dst_base = wid * per_w
        idx_b, rows_b, sem_b = (idx_v0, idx_v1), (rows_v0, rows_v1), (sem0, sem1)

        def start(i):
            b = i % 2
            pltpu.sync_copy(idx_hbm.at[pl.ds(src_base + i * chunk, chunk)],
                            idx_b[b])
            return pltpu.async_copy(table_hbm.at[idx_b[b]], rows_b[b], sem_b[b])

        pending = start(0)
        for i in range(steps):
            nxt = start(i + 1) if i + 1 < steps else None
            pending.wait()
            pltpu.sync_copy(rows_b[i % 2],
                            out_hbm.at[pl.ds(dst_base + i * chunk, chunk)])
            pending = nxt

    return gather_kernel(table, idx)


def kernel(x, neighbors, max_iter, last_layer):
    del max_iter, last_layer
    n, d = x.shape
    m = neighbors.shape[0] // n
    table_body, nbr_t = _prep_call(x, neighbors.reshape(n, m))
    table = jnp.concatenate(
        [table_body, jnp.zeros((8, d // 2), dtype=jnp.int32)], axis=0)
    na = 1600
    x32 = table_body.reshape(n // 2, d)
    us = []
    for lo, hi, chunk in ((0, na, 800), (na, n, 840)):
        nbr_c = nbr_t[:, lo:hi].reshape(m * (hi - lo))
        z32 = _gather_call(table, nbr_c, hi - lo, 0, hi - lo, chunk)
        z32v = z32.reshape(m, (hi - lo) // 2, d)
        us.append(_routing_call(z32v, x32[lo // 2: hi // 2]))
    return jnp.concatenate(us, axis=0).reshape(n, d)


# speedup vs baseline: 1.0115x; 1.0004x over previous
import functools

import jax
import jax.numpy as jnp
from jax import lax
from jax.experimental import pallas as pl
from jax.experimental.pallas import tpu as pltpu
from jax.experimental.pallas import tpu_sc as plsc

_K = 8
_DD = 16
_NC, _NS = 2, 16
_NW = _NC * _NS
_ITERS = 6
_LOG2E = 1.4426950408889634


def _seg_matrix(d, dtype):
    lane = lax.broadcasted_iota(jnp.int32, (d, _K), 0)
    cap = lax.broadcasted_iota(jnp.int32, (d, _K), 1)
    return (lane // _DD == cap).astype(dtype)


def _seg2(d, lo, dtype):
    lane = lax.broadcasted_iota(jnp.int32, (d, 2 * _K), 0)
    col = lax.broadcasted_iota(jnp.int32, (d, 2 * _K), 1)
    cap = (lane % 64) // _DD + (0 if lo else _K // 2) + _K * (lane // 64)
    return (cap == col).astype(dtype)


def _unpack_lo(v):
    return lax.bitcast_convert_type(v << 16, jnp.float32)


def _unpack_hi(v):
    return lax.bitcast_convert_type(v & jnp.int32(-65536), jnp.float32)


def _rne_bf16_bits(f):
    b = lax.bitcast_convert_type(f, jnp.int32)
    return (b + 0x7FFF + ((b >> 16) & 1)) >> 16


def _prep_body(x_ref, nbr_ref, t_ref, nt_ref):
    x = x_ref[...]
    d = x.shape[-1]
    s_mat = _seg_matrix(d, x.dtype)
    sq = jnp.dot(x * x, s_mat, preferred_element_type=jnp.float32)
    scale = lax.rsqrt(jnp.maximum(sq, 1e-24))
    xn = x * jnp.dot(scale, s_mat.T, preferred_element_type=jnp.float32)
    lo = _rne_bf16_bits(xn[:, : d // 2])
    hi = _rne_bf16_bits(xn[:, d // 2:])
    t_ref[...] = (hi << 16) | (lo & 0xFFFF)

    @pl.when(pl.program_id(0) == 0)
    def _():
        nt_ref[...] = nbr_ref[...].T


def _routing_body(z_ref, x_ref, o_ref):
    z32 = z_ref[...]
    x32 = x_ref[...]
    m, bh, d = z32.shape
    z_lo, z_hi = _unpack_lo(z32), _unpack_hi(z32)
    x_lo, x_hi = _unpack_lo(x32), _unpack_hi(x32)
    s_lo = _seg2(d, True, jnp.float32)
    s_hi = _seg2(d, False, jnp.float32)
    s16_lo = s_lo.astype(jnp.bfloat16)
    s16_hi = s_hi.astype(jnp.bfloat16)
    sl16_lo = (s_lo * _LOG2E).astype(jnp.bfloat16)
    sl16_hi = (s_hi * _LOG2E).astype(jnp.bfloat16)
    r16 = lax.broadcasted_iota(jnp.int32, (2 * _K, 2 * _K), 0)
    c16 = lax.broadcasted_iota(jnp.int32, (2 * _K, 2 * _K), 1)
    b16 = (r16 // _K == c16 // _K).astype(jnp.float32)
    zf_lo = z_lo.reshape(m * bh, d)
    zf_hi = z_hi.reshape(m * bh, d)

    def cap_norm(u_lo, u_hi):
        sq = (jnp.dot(u_lo * u_lo, s_lo, preferred_element_type=jnp.float32)
              + jnp.dot(u_hi * u_hi, s_hi, preferred_element_type=jnp.float32))
        scale = lax.rsqrt(jnp.maximum(sq, 1e-24))
        return (u_lo * jnp.dot(scale, s_lo.T,
                               preferred_element_type=jnp.float32),
                u_hi * jnp.dot(scale, s_hi.T,
                               preferred_element_type=jnp.float32))

    u_lo = jnp.sum(z_lo, axis=0) * (1.0 / _K) + x_lo
    u_hi = jnp.sum(z_hi, axis=0) * (1.0 / _K) + x_hi
    u_lo, u_hi = cap_norm(u_lo, u_hi)
    for it in range(1, _ITERS):
        t_lo = (z_lo * u_lo[None, :, :]).reshape(m * bh, d).astype(jnp.bfloat16)
        t_hi = (z_hi * u_hi[None, :, :]).reshape(m * bh, d).astype(jnp.bfloat16)
        p = (jnp.dot(t_lo, sl16_lo, preferred_element_type=jnp.float32)
             + jnp.dot(t_hi, sl16_hi, preferred_element_type=jnp.float32))
        e = jnp.exp2(p)
        den = jnp.dot(e, b16, preferred_element_type=jnp.float32)
        pn = (e / den).astype(jnp.bfloat16)
        pe_lo = jnp.dot(pn, s16_lo.T, preferred_element_type=jnp.float32)
        pe_hi = jnp.dot(pn, s16_hi.T, preferred_element_type=jnp.float32)
        u_lo = jnp.sum((zf_lo * pe_lo).reshape(m, bh, d), axis=0) + x_lo
        u_hi = jnp.sum((zf_hi * pe_hi).reshape(m, bh, d), axis=0) + x_hi
        if it < _ITERS - 1:
            u_lo, u_hi = cap_norm(u_lo, u_hi)
    h = d // 2
    o_ref[...] = jnp.concatenate(
        [u_lo[:, :h], u_hi[:, :h], u_lo[:, h:], u_hi[:, h:]], axis=1)


def _prep_call(x, nbr2):
    n, d = x.shape
    m = nbr2.shape[1]
    bn = 2000
    return pl.pallas_call(
        _prep_body,
        grid=(n // bn,),
        in_specs=[pl.BlockSpec((bn, d), lambda i: (i, 0)),
                  pl.BlockSpec((n, m), lambda i: (0, 0))],
        out_specs=[pl.BlockSpec((bn, d // 2), lambda i: (i, 0)),
                   pl.BlockSpec((m, n), lambda i: (0, 0))],
        out_shape=[jax.ShapeDtypeStruct((n, d // 2), jnp.int32),
                   jax.ShapeDtypeStruct((m, n), nbr2.dtype)],
    )(x, nbr2)


def _routing_call(z32v, x32v):
    m, n2, d = z32v.shape
    bh = 200
    return pl.pallas_call(
        _routing_body,
        grid=(n2 // bh,),
        in_specs=[
            pl.BlockSpec((m, bh, d), lambda i: (0, i, 0)),
            pl.BlockSpec((bh, d), lambda i: (i, 0)),
        ],
        out_specs=pl.BlockSpec((bh, 2 * d), lambda i: (i, 0)),
        out_shape=jax.ShapeDtypeStruct((n2, 2 * d), jnp.float32),
        compiler_params=pltpu.CompilerParams(
            vmem_limit_bytes=120 * 1024 * 1024),
    )(z32v, x32v)


def _gather_call(table, idx, chunk):
    e_total = idx.shape[0]
    per_w = e_total // _NW
    steps = per_w // chunk
    d = table.shape[1]
    mesh = plsc.VectorSubcoreMesh(core_axis_name="c", subcore_axis_name="s")

    @functools.partial(
        pl.kernel,
        out_type=jax.ShapeDtypeStruct((e_total, d), table.dtype),
        mesh=mesh,
        scratch_types=[
            pltpu.VMEM((chunk,), jnp.int32),
            pltpu.VMEM((chunk,), jnp.int32),
            pltpu.VMEM((chunk, d), table.dtype),
            pltpu.VMEM((chunk, d), table.dtype),
            pltpu.SemaphoreType.DMA,
            pltpu.SemaphoreType.DMA,
        ],
        compiler_params=pltpu.CompilerParams(use_tc_tiling_on_sc=False),
    )
    def gather_kernel(table_hbm, idx_hbm, out_hbm,
                      idx_v0, idx_v1, rows_v0, rows_v1, sem0, sem1):
        wid = lax.axis_index("s") * _NC + lax.axis_index("c")
        base = wid * per_w
        idx_b, rows_b, sem_b = (idx_v0, idx_v1), (rows_v0, rows_v1), (sem0, sem1)

        def start(i):
            b = i % 2
            pltpu.sync_copy(idx_hbm.at[pl.ds(base + i * chunk, chunk)],
                            idx_b[b])
            return pltpu.async_copy(table_hbm.at[idx_b[b]], rows_b[b], sem_b[b])

        pending = start(0)
        for i in range(steps):
            nxt = start(i + 1) if i + 1 < steps else None
            pending.wait()
            pltpu.sync_copy(rows_b[i % 2],
                            out_hbm.at[pl.ds(base + i * chunk, chunk)])
            pending = nxt

    return gather_kernel(table, idx)


def kernel(x, neighbors, max_iter, last_layer):
    del max_iter, last_layer
    n, d = x.shape
    m = neighbors.shape[0] // n
    table_body, nbr_t = _prep_call(x, neighbors.reshape(n, m))
    table = jnp.concatenate(
        [table_body, jnp.zeros((8, d // 2), dtype=jnp.int32)], axis=0)
    na = 1600
    x32 = table_body.reshape(n // 2, d)
    us = []
    for lo, hi, chunk in ((0, na, 800), (na, n, 840)):
        nbr_c = nbr_t[:, lo:hi].reshape(m * (hi - lo))
        z32 = _gather_call(table, nbr_c, chunk)
        z32v = z32.reshape(m, (hi - lo) // 2, d)
        us.append(_routing_call(z32v, x32[lo // 2: hi // 2]))
    return jnp.concatenate(us, axis=0).reshape(n, d)
